# baseline (speedup 1.0000x reference)
import functools

import jax
import jax.numpy as jnp
from jax import lax
from jax.experimental import pallas as pl
from jax.experimental.pallas import tpu as pltpu
from jax.experimental.pallas import tpu_sc as plsc

_CHUNK = 128
_W = 128


@functools.cache
def _make_kernel(B, D_u, D_m, S, N_u, N_m):
    info = plsc.get_sparse_core_info()
    NC, NS = info.num_cores, info.num_subcores
    NW = NC * NS
    assert B % (NW * _CHUNK) == 0
    b_per_w = B // NW
    n_ch = b_per_w // _CHUNK
    D = D_u + D_m
    mesh = plsc.VectorSubcoreMesh(core_axis_name="c", subcore_axis_name="s")

    @functools.partial(
        pl.kernel,
        mesh=mesh,
        out_type=jax.ShapeDtypeStruct((B, D), jnp.float32),
        compiler_params=pltpu.CompilerParams(use_tc_tiling_on_sc=True),
        scratch_types=[
            pltpu.VMEM((n_ch, _CHUNK), jnp.int32),
            pltpu.VMEM((n_ch, _CHUNK), jnp.int32),
            pltpu.VMEM((n_ch, _CHUNK), jnp.int32),
            pltpu.VMEM((n_ch, _CHUNK), jnp.int32),
            pltpu.VMEM((_CHUNK, _W), jnp.float32),
            pltpu.VMEM((_CHUNK, _W), jnp.float32),
            pltpu.VMEM((_CHUNK, _W), jnp.float32),
            pltpu.VMEM((b_per_w, D), jnp.float32),
            pltpu.SemaphoreType.DMA,
        ],
    )
    def k(uid_hbm, mid_hbm, uta_hbm, utb_hbm, mt_hbm, out_hbm,
          uids, mids, ixa, ixb, tua, tub, tm, comb, sem):
        wid = lax.axis_index("s") * NC + lax.axis_index("c")
        base = wid * b_per_w
        for c in range(n_ch):
            pltpu.sync_copy(uid_hbm.at[pl.ds(base + c * _CHUNK, _CHUNK)],
                            uids.at[c])
            pltpu.sync_copy(mid_hbm.at[pl.ds(base + c * _CHUNK, _CHUNK)],
                            mids.at[c])
        for c in range(n_ch):
            for g in range(_CHUNK // 16):
                s = pl.ds(g * 16, 16)
                v = uids[c, s]
                ixa[c, s] = jnp.minimum(v, S - 1)
                ixb[c, s] = jnp.clip(v - S, 0, N_u - S - 1)
        for c in range(n_ch):
            cps = [
                pltpu.async_copy(uta_hbm.at[ixa.at[c]], tua, sem),
                pltpu.async_copy(utb_hbm.at[ixb.at[c]], tub, sem),
                pltpu.async_copy(mt_hbm.at[mids.at[c]], tm, sem),
            ]
            for cp in cps:
                cp.wait()

            @pl.loop(0, _CHUNK // 16)
            def _mg(g):
                r0 = g * 16
                idv = uids[c, pl.ds(r0, 16)]
                for l in range(16):
                    rr = r0 + l
                    r = c * _CHUNK + rr
                    in_a = idv[l] < S

                    @pl.when(in_a)
                    def _a():
                        comb[r, pl.ds(0, 16)] = tua[rr, pl.ds(0, 16)]
                        comb[r, pl.ds(15, 16)] = tua[rr, pl.ds(15, 16)]

                    @pl.when(jnp.logical_not(in_a))
                    def _b():
                        comb[r, pl.ds(0, 16)] = tub[rr, pl.ds(0, 16)]
                        comb[r, pl.ds(15, 16)] = tub[rr, pl.ds(15, 16)]

                    comb[r, pl.ds(D_u, 16)] = tm[rr, pl.ds(0, 16)]
                    comb[r, pl.ds(D_u + 1, 16)] = tm[rr, pl.ds(1, 16)]

        pltpu.sync_copy(comb, out_hbm.at[pl.ds(base, b_per_w)])

    return k


def kernel(user_ids, movie_ids, user_table, movie_table):
    B = user_ids.shape[0]
    N_u, D_u = user_table.shape
    N_m, D_m = movie_table.shape
    S = (N_u // 2 // 128) * 128
    uta = jnp.pad(user_table[:S], ((0, 0), (0, _W - D_u)))
    utb = jnp.pad(user_table[S:], ((0, 0), (0, _W - D_u)))
    mt128 = jnp.pad(movie_table, ((0, 0), (0, _W - D_m)))
    k = _make_kernel(B, D_u, D_m, S, N_u, N_m)
    return k(user_ids, movie_ids, uta, utb, mt128)


# speedup vs baseline: 1.6983x; 1.6983x over previous
import functools

import jax
import jax.numpy as jnp
from jax import lax
from jax.experimental import pallas as pl
from jax.experimental.pallas import tpu as pltpu
from jax.experimental.pallas import tpu_sc as plsc

_CHUNK = 128
_W = 128


@functools.cache
def _make_kernel(B, D_u, D_m):
    info = plsc.get_sparse_core_info()
    NC, NS = info.num_cores, info.num_subcores
    NW = NC * NS
    assert B % (NW * _CHUNK) == 0
    b_per_w = B // NW
    n_ch = b_per_w // _CHUNK
    D = D_u + D_m
    mesh = plsc.VectorSubcoreMesh(core_axis_name="c", subcore_axis_name="s")

    @functools.partial(
        pl.kernel,
        mesh=mesh,
        out_type=jax.ShapeDtypeStruct((B, D), jnp.float32),
        compiler_params=pltpu.CompilerParams(use_tc_tiling_on_sc=True),
        scratch_types=[
            pltpu.VMEM((n_ch, _CHUNK), jnp.int32),
            pltpu.VMEM((n_ch, _CHUNK), jnp.int32),
            pltpu.VMEM((_CHUNK, _W), jnp.float32),
            pltpu.VMEM((_CHUNK, _W), jnp.float32),
            pltpu.VMEM((b_per_w, D), jnp.float32),
            pltpu.SemaphoreType.DMA,
        ],
    )
    def k(uid_hbm, mid_hbm, ut_hbm, mt_hbm, out_hbm,
          uids, mids, tu, tm, comb, sem):
        wid = lax.axis_index("s") * NC + lax.axis_index("c")
        base = wid * b_per_w
        for c in range(n_ch):
            pltpu.sync_copy(uid_hbm.at[pl.ds(base + c * _CHUNK, _CHUNK)],
                            uids.at[c])
            pltpu.sync_copy(mid_hbm.at[pl.ds(base + c * _CHUNK, _CHUNK)],
                            mids.at[c])
        for c in range(n_ch):
            cu = pltpu.async_copy(ut_hbm.at[uids.at[c]], tu, sem)
            cm = pltpu.async_copy(mt_hbm.at[mids.at[c]], tm, sem)
            cu.wait()
            cm.wait()

            @pl.loop(0, _CHUNK)
            def _merge(rr):
                r = c * _CHUNK + rr
                comb[r, pl.ds(0, 16)] = tu[rr, pl.ds(0, 16)]
                comb[r, pl.ds(15, 16)] = tu[rr, pl.ds(15, 16)]
                comb[r, pl.ds(D_u, 16)] = tm[rr, pl.ds(0, 16)]
                comb[r, pl.ds(D_u + 1, 16)] = tm[rr, pl.ds(1, 16)]

        pltpu.sync_copy(comb, out_hbm.at[pl.ds(base, b_per_w)])

    return k


def kernel(user_ids, movie_ids, user_table, movie_table):
    B = user_ids.shape[0]
    N_u, D_u = user_table.shape
    N_m, D_m = movie_table.shape
    ut128 = jnp.pad(user_table, ((0, 0), (0, _W - D_u)))
    mt128 = jnp.pad(movie_table, ((0, 0), (0, _W - D_m)))
    k = _make_kernel(B, D_u, D_m)
    return k(user_ids, movie_ids, ut128, mt128)
